# baseline (speedup 1.0000x reference)
import functools

import jax
import jax.numpy as jnp
from jax.experimental import pallas as pl
from jax.experimental.pallas import tpu as pltpu

_K = 10
_THRESH = 0.1
_SUBH = 32


def _composite_kernel(s_ref, alpha_hbm, color_hbm, canvas_ref, den_ref,
                      abuf, cbuf, T_ref, C_ref, D_ref, cnt_ref, sem,
                      *, ch, num_chunks, w):
    b = pl.program_id(0)

    T_ref[...] = jnp.ones_like(T_ref)
    C_ref[...] = jnp.zeros_like(C_ref)
    D_ref[...] = jnp.zeros_like(D_ref)
    cnt_ref[...] = jnp.zeros_like(cnt_ref)

    def copies(j, slot, descending):
        start = ((num_chunks - 1 - j) if descending else j) * ch
        return (
            pltpu.make_async_copy(alpha_hbm.at[b, pl.ds(start, ch)],
                                  abuf.at[slot], sem.at[slot, 0]),
            pltpu.make_async_copy(color_hbm.at[b, pl.ds(start, ch)],
                                  cbuf.at[slot], sem.at[slot, 1]),
        )

    def start_copies(j, slot, descending):
        for c in copies(j, slot, descending):
            c.start()

    def wait_copies(j, slot, descending):
        for c in copies(j, slot, descending):
            c.wait()

    def chunk_compute(slot, base, descending):
        nsub = w // _SUBH

        def outer(si, _):
            rows = pl.ds(si * _SUBH, _SUBH)
            T = T_ref[rows, :]
            cnt = cnt_ref[rows, :]
            C0 = C_ref[0, rows, :]
            C1 = C_ref[1, rows, :]
            C2 = C_ref[2, rows, :]
            D = D_ref[rows, :]

            def sbody(i, carry):
                T, cnt, C0, C1, C2, D = carry
                idx = (ch - 1 - i) if descending else i
                a = abuf[slot, idx, 0, rows, :]
                vis = a > _THRESH
                if not descending:
                    vis = jnp.logical_not(vis)
                take = vis & (cnt < _K)
                w_ = jnp.where(take, T * a, 0.0)
                C0 = C0 + w_ * cbuf[slot, idx, 0, rows, :]
                C1 = C1 + w_ * cbuf[slot, idx, 1, rows, :]
                C2 = C2 + w_ * cbuf[slot, idx, 2, rows, :]
                if descending:
                    D = D + w_ * s_ref[0, base + idx, 0, 0]
                T = T - w_
                cnt = cnt + take.astype(jnp.int32)
                return T, cnt, C0, C1, C2, D

            T, cnt, C0, C1, C2, D = jax.lax.fori_loop(
                0, ch, sbody, (T, cnt, C0, C1, C2, D))

            T_ref[rows, :] = T
            cnt_ref[rows, :] = cnt
            C_ref[0, rows, :] = C0
            C_ref[1, rows, :] = C1
            C_ref[2, rows, :] = C2
            D_ref[rows, :] = D
            return 0

        jax.lax.fori_loop(0, nsub, outer, 0)

    def run_pass(descending):
        start_copies(0, 0, descending)

        def cond(state):
            j, done = state
            return jnp.logical_and(jnp.logical_not(done), j < num_chunks)

        def body(state):
            j, _ = state
            slot = jax.lax.rem(j, 2)

            @pl.when(j + 1 < num_chunks)
            def _prefetch():
                start_copies(j + 1, 1 - slot, descending)

            wait_copies(j, slot, descending)
            base = ((num_chunks - 1 - j) if descending else j) * ch
            chunk_compute(slot, base, descending)
            done = jnp.min(cnt_ref[...]) >= _K
            return j + 1, done

        jf, _ = jax.lax.while_loop(cond, body,
                                   (jnp.int32(0), jnp.bool_(False)))

        @pl.when(jf < num_chunks)
        def _drain():
            wait_copies(jf, jax.lax.rem(jf, 2), descending)

    run_pass(descending=True)

    @pl.when(jnp.min(cnt_ref[...]) < _K)
    def _tie_fill():
        run_pass(descending=False)

    T = T_ref[...]
    canvas_ref[0, 0] = C_ref[0] + T
    canvas_ref[0, 1] = C_ref[1] + T
    canvas_ref[0, 2] = C_ref[2] + T
    den_ref[0, 0] = D_ref[...] + T


@jax.jit
def kernel(color_stroke, alpha, params):
    b, n = color_stroke.shape[0], color_stroke.shape[1]
    w = color_stroke.shape[-1]
    ch = 8
    num_chunks = n // ch

    s = (params[:, :, 2] * params[:, :, 3]).reshape(b, n, 1, 1)

    kfn = functools.partial(_composite_kernel, ch=ch, num_chunks=num_chunks,
                            w=w)

    canvas, den = pl.pallas_call(
        kfn,
        grid=(b,),
        in_specs=[
            pl.BlockSpec((1, n, 1, 1), lambda bi: (bi, 0, 0, 0)),
            pl.BlockSpec(memory_space=pl.ANY),
            pl.BlockSpec(memory_space=pl.ANY),
        ],
        out_specs=[
            pl.BlockSpec((1, 3, w, w), lambda bi: (bi, 0, 0, 0)),
            pl.BlockSpec((1, 1, w, w), lambda bi: (bi, 0, 0, 0)),
        ],
        out_shape=[
            jax.ShapeDtypeStruct((b, 3, w, w), jnp.float32),
            jax.ShapeDtypeStruct((b, 1, w, w), jnp.float32),
        ],
        scratch_shapes=[
            pltpu.VMEM((2, ch, 1, w, w), jnp.float32),
            pltpu.VMEM((2, ch, 3, w, w), jnp.float32),
            pltpu.VMEM((w, w), jnp.float32),
            pltpu.VMEM((3, w, w), jnp.float32),
            pltpu.VMEM((w, w), jnp.float32),
            pltpu.VMEM((w, w), jnp.int32),
            pltpu.SemaphoreType.DMA((2, 2)),
        ],
        compiler_params=pltpu.CompilerParams(
            dimension_semantics=("arbitrary",),
        ),
    )(s, alpha, color_stroke)

    return (canvas, den)


# speedup vs baseline: 1.3744x; 1.3744x over previous
import functools

import jax
import jax.numpy as jnp
from jax.experimental import pallas as pl
from jax.experimental.pallas import tpu as pltpu

_K = 10
_THRESH = 0.1
_SUBH = 32


def _composite_kernel(s_ref, alpha_hbm, color_hbm, canvas_ref, den_ref,
                      abuf, cbuf, T_ref, C_ref, D_ref, cnt_ref, sem,
                      *, ch, num_chunks, w):
    b = pl.program_id(0)

    T_ref[...] = jnp.ones_like(T_ref)
    C_ref[...] = jnp.zeros_like(C_ref)
    D_ref[...] = jnp.zeros_like(D_ref)
    cnt_ref[...] = jnp.zeros_like(cnt_ref)

    def copies(j, slot, descending):
        start = ((num_chunks - 1 - j) if descending else j) * ch
        return (
            pltpu.make_async_copy(alpha_hbm.at[b, pl.ds(start, ch)],
                                  abuf.at[slot], sem.at[slot, 0]),
            pltpu.make_async_copy(color_hbm.at[b, pl.ds(start, ch)],
                                  cbuf.at[slot], sem.at[slot, 1]),
        )

    def start_copies(j, slot, descending):
        for c in copies(j, slot, descending):
            c.start()

    def wait_copies(j, slot, descending):
        for c in copies(j, slot, descending):
            c.wait()

    def chunk_compute(slot, base, descending):
        nsub = w // _SUBH

        for si in range(nsub):
            rows = pl.ds(si * _SUBH, _SUBH)
            T = T_ref[rows, :]
            cnt = cnt_ref[rows, :]
            C0 = C_ref[0, rows, :]
            C1 = C_ref[1, rows, :]
            C2 = C_ref[2, rows, :]
            D = D_ref[rows, :]

            for i in range(ch):
                idx = (ch - 1 - i) if descending else i
                a = abuf[slot, idx, 0, rows, :]
                vis = a > _THRESH
                if not descending:
                    vis = jnp.logical_not(vis)
                take = vis & (cnt < _K)
                w_ = jnp.where(take, T * a, 0.0)
                C0 = C0 + w_ * cbuf[slot, idx, 0, rows, :]
                C1 = C1 + w_ * cbuf[slot, idx, 1, rows, :]
                C2 = C2 + w_ * cbuf[slot, idx, 2, rows, :]
                if descending:
                    D = D + w_ * s_ref[0, base + idx, 0, 0]
                T = T - w_
                cnt = cnt + take.astype(jnp.int32)

            T_ref[rows, :] = T
            cnt_ref[rows, :] = cnt
            C_ref[0, rows, :] = C0
            C_ref[1, rows, :] = C1
            C_ref[2, rows, :] = C2
            D_ref[rows, :] = D

    def run_pass(descending):
        start_copies(0, 0, descending)

        def cond(state):
            j, done = state
            return jnp.logical_and(jnp.logical_not(done), j < num_chunks)

        def body(state):
            j, _ = state
            slot = jax.lax.rem(j, 2)

            @pl.when(j + 1 < num_chunks)
            def _prefetch():
                start_copies(j + 1, 1 - slot, descending)

            wait_copies(j, slot, descending)
            base = ((num_chunks - 1 - j) if descending else j) * ch
            chunk_compute(slot, base, descending)
            done = jnp.min(cnt_ref[...]) >= _K
            return j + 1, done

        jf, _ = jax.lax.while_loop(cond, body,
                                   (jnp.int32(0), jnp.bool_(False)))

        @pl.when(jf < num_chunks)
        def _drain():
            wait_copies(jf, jax.lax.rem(jf, 2), descending)

    run_pass(descending=True)

    @pl.when(jnp.min(cnt_ref[...]) < _K)
    def _tie_fill():
        run_pass(descending=False)

    T = T_ref[...]
    canvas_ref[0, 0] = C_ref[0] + T
    canvas_ref[0, 1] = C_ref[1] + T
    canvas_ref[0, 2] = C_ref[2] + T
    den_ref[0, 0] = D_ref[...] + T


@jax.jit
def kernel(color_stroke, alpha, params):
    b, n = color_stroke.shape[0], color_stroke.shape[1]
    w = color_stroke.shape[-1]
    ch = 8
    num_chunks = n // ch

    s = (params[:, :, 2] * params[:, :, 3]).reshape(b, n, 1, 1)

    kfn = functools.partial(_composite_kernel, ch=ch, num_chunks=num_chunks,
                            w=w)

    canvas, den = pl.pallas_call(
        kfn,
        grid=(b,),
        in_specs=[
            pl.BlockSpec((1, n, 1, 1), lambda bi: (bi, 0, 0, 0)),
            pl.BlockSpec(memory_space=pl.ANY),
            pl.BlockSpec(memory_space=pl.ANY),
        ],
        out_specs=[
            pl.BlockSpec((1, 3, w, w), lambda bi: (bi, 0, 0, 0)),
            pl.BlockSpec((1, 1, w, w), lambda bi: (bi, 0, 0, 0)),
        ],
        out_shape=[
            jax.ShapeDtypeStruct((b, 3, w, w), jnp.float32),
            jax.ShapeDtypeStruct((b, 1, w, w), jnp.float32),
        ],
        scratch_shapes=[
            pltpu.VMEM((2, ch, 1, w, w), jnp.float32),
            pltpu.VMEM((2, ch, 3, w, w), jnp.float32),
            pltpu.VMEM((w, w), jnp.float32),
            pltpu.VMEM((3, w, w), jnp.float32),
            pltpu.VMEM((w, w), jnp.float32),
            pltpu.VMEM((w, w), jnp.int32),
            pltpu.SemaphoreType.DMA((2, 2)),
        ],
        compiler_params=pltpu.CompilerParams(
            dimension_semantics=("arbitrary",),
        ),
    )(s, alpha, color_stroke)

    return (canvas, den)
